# baseline (speedup 1.0000x reference)
import functools

import jax
import jax.numpy as jnp
from jax import lax
from jax.experimental import pallas as pl
from jax.experimental.pallas import tpu as pltpu
from jax.experimental.pallas import tpu_sc as plsc

NC, NS = 2, 16
NW = NC * NS
B = 16384
D = 64
BPW = B // NW
NIDX = BPW // 128


def _sc_body(idx_hbm, wg_hbm, wm_hbm, g_out, m0_out, m1_out,
             idx0_v, idx1_v, r0_v, r1_v, m0_v, sem_g, sem_m):
    wid = lax.axis_index("s") * NC + lax.axis_index("c")
    base = wid * BPW
    pltpu.sync_copy(idx_hbm.at[0, wid], idx0_v)
    pltpu.sync_copy(idx_hbm.at[1, wid], idx1_v)

    gmf_dmas = []
    for j in range(NIDX):
        gmf_dmas.append(pltpu.async_copy(
            wg_hbm.at[idx0_v.at[j]], r0_v.at[pl.ds(j * 128, 128)], sem_g))
        gmf_dmas.append(pltpu.async_copy(
            wg_hbm.at[idx1_v.at[j]], r1_v.at[pl.ds(j * 128, 128)], sem_g))
    m0_dmas = []
    for j in range(NIDX):
        m0_dmas.append(pltpu.async_copy(
            wm_hbm.at[idx0_v.at[j]], m0_v.at[pl.ds(j * 128, 128)], sem_m))
    for d in gmf_dmas:
        d.wait()

    def prod_row(r, _):
        for c in range(D // 16):
            r0_v[r, pl.ds(c * 16, 16)] = (
                r0_v[r, pl.ds(c * 16, 16)] * r1_v[r, pl.ds(c * 16, 16)])
        return _
    lax.fori_loop(0, BPW, prod_row, 0, unroll=2)
    pltpu.sync_copy(r0_v, g_out.at[pl.ds(base, BPW)])

    m1_dmas = []
    for j in range(NIDX):
        m1_dmas.append(pltpu.async_copy(
            wm_hbm.at[idx1_v.at[j]], r1_v.at[pl.ds(j * 128, 128)], sem_g))
    for d in m0_dmas:
        d.wait()
    pltpu.sync_copy(m0_v, m0_out.at[pl.ds(base, BPW)])
    for d in m1_dmas:
        d.wait()
    pltpu.sync_copy(r1_v, m1_out.at[pl.ds(base, BPW)])


@jax.jit
def _sc_gather(idx, wg, wm):
    mesh = plsc.VectorSubcoreMesh(core_axis_name="c", subcore_axis_name="s")
    f = pl.kernel(
        _sc_body,
        out_type=(
            jax.ShapeDtypeStruct((B, D), jnp.float32),
            jax.ShapeDtypeStruct((B, D), jnp.float32),
            jax.ShapeDtypeStruct((B, D), jnp.float32),
        ),
        mesh=mesh,
        scratch_types=[
            pltpu.VMEM((NIDX, 128), jnp.int32),
            pltpu.VMEM((NIDX, 128), jnp.int32),
            pltpu.VMEM((BPW, D), jnp.float32),
            pltpu.VMEM((BPW, D), jnp.float32),
            pltpu.VMEM((BPW, D), jnp.float32),
            pltpu.SemaphoreType.DMA,
            pltpu.SemaphoreType.DMA,
        ],
        compiler_params=pltpu.CompilerParams(use_tc_tiling_on_sc=False),
    )
    return f(idx, wg, wm)


BLK = 2048


def _tc_body(g_ref, m0_ref, m1_ref, w0a, w0b, b0, w1t, b1, w2t, b2,
             wcg, wch, bc, out_ref):
    f32 = jnp.float32
    h = jnp.dot(m0_ref[...], w0a[...], preferred_element_type=f32)
    h = h + jnp.dot(m1_ref[...], w0b[...], preferred_element_type=f32)
    h = jnp.maximum(h + b0[...], 0.0)
    h = jnp.maximum(jnp.dot(h, w1t[...], preferred_element_type=f32) + b1[...], 0.0)
    h = jnp.maximum(jnp.dot(h, w2t[...], preferred_element_type=f32) + b2[...], 0.0)
    logit = (jnp.dot(h, wch[...], preferred_element_type=f32)
             + jnp.dot(g_ref[...], wcg[...], preferred_element_type=f32)
             + bc[...])
    out_ref[...] = logit


@jax.jit
def _tc_mlp(g, m0, m1, w0a, w0b, b0, w1t, b1, w2t, b2, wcg, wch, bc):
    full = lambda shape: pl.BlockSpec(shape, lambda i: (0, 0))
    return pl.pallas_call(
        _tc_body,
        grid=(B // BLK,),
        in_specs=[
            pl.BlockSpec((BLK, D), lambda i: (i, 0)),
            pl.BlockSpec((BLK, D), lambda i: (i, 0)),
            pl.BlockSpec((BLK, D), lambda i: (i, 0)),
            full(w0a.shape), full(w0b.shape), full(b0.shape),
            full(w1t.shape), full(b1.shape),
            full(w2t.shape), full(b2.shape),
            full(wcg.shape), full(wch.shape), full(bc.shape),
        ],
        out_specs=pl.BlockSpec((BLK, 1), lambda i: (i, 0)),
        out_shape=jax.ShapeDtypeStruct((B, 1), jnp.float32),
    )(g, m0, m1, w0a, w0b, b0, w1t, b1, w2t, b2, wcg, wch, bc)


@jax.jit
def _pipeline(sparse_features, W_gmf, W_mlp, Wl0, bl0, Wl1, bl1, Wl2, bl2, Wc, bc):
    idx = sparse_features.astype(jnp.int32).T.reshape(2, NW, NIDX, 128)
    g, m0, m1 = _sc_gather(idx, W_gmf, W_mlp)
    return _tc_mlp(
        g, m0, m1,
        Wl0[:, :D].T, Wl0[:, D:].T, bl0.reshape(1, -1),
        Wl1.T, bl1.reshape(1, -1),
        Wl2.T, bl2.reshape(1, -1),
        Wc[:, :D].T, Wc[:, D:].T, bc.reshape(1, 1),
    )


def kernel(sparse_features, W_gmf, W_mlp, Wl0, bl0, Wl1, bl1, Wl2, bl2, Wc, bc):
    return _pipeline(sparse_features, W_gmf, W_mlp, Wl0, bl0, Wl1, bl1, Wl2,
                     bl2, Wc, bc)


# speedup vs baseline: 1.0153x; 1.0153x over previous
import functools

import jax
import jax.numpy as jnp
from jax import lax
from jax.experimental import pallas as pl
from jax.experimental.pallas import tpu as pltpu
from jax.experimental.pallas import tpu_sc as plsc

NC, NS = 2, 16
NW = NC * NS
B = 16384
D = 64
BPW = B // NW
NIDX = BPW // 128


def _sc_gmf_body(idx_hbm, wg_hbm, g_out, idx0_v, idx1_v, r0_v, r1_v, sem_g):
    wid = lax.axis_index("s") * NC + lax.axis_index("c")
    base = wid * BPW
    pltpu.sync_copy(idx_hbm.at[0, wid], idx0_v)
    pltpu.sync_copy(idx_hbm.at[1, wid], idx1_v)
    dmas = []
    for j in range(NIDX):
        dmas.append(pltpu.async_copy(
            wg_hbm.at[idx0_v.at[j]], r0_v.at[pl.ds(j * 128, 128)], sem_g))
        dmas.append(pltpu.async_copy(
            wg_hbm.at[idx1_v.at[j]], r1_v.at[pl.ds(j * 128, 128)], sem_g))
    for d in dmas:
        d.wait()

    def prod_row(r, _):
        for c in range(D // 16):
            r0_v[r, pl.ds(c * 16, 16)] = (
                r0_v[r, pl.ds(c * 16, 16)] * r1_v[r, pl.ds(c * 16, 16)])
        return _
    lax.fori_loop(0, BPW, prod_row, 0, unroll=2)
    pltpu.sync_copy(r0_v, g_out.at[pl.ds(base, BPW)])


def _sc_mlp_body(idx_hbm, wm_hbm, m0_out, m1_out, idx0_v, idx1_v, m0_v, m1_v,
                 sem_m):
    wid = lax.axis_index("s") * NC + lax.axis_index("c")
    base = wid * BPW
    pltpu.sync_copy(idx_hbm.at[0, wid], idx0_v)
    pltpu.sync_copy(idx_hbm.at[1, wid], idx1_v)
    dmas = []
    for j in range(NIDX):
        dmas.append(pltpu.async_copy(
            wm_hbm.at[idx0_v.at[j]], m0_v.at[pl.ds(j * 128, 128)], sem_m))
        dmas.append(pltpu.async_copy(
            wm_hbm.at[idx1_v.at[j]], m1_v.at[pl.ds(j * 128, 128)], sem_m))
    for d in dmas:
        d.wait()
    pltpu.sync_copy(m0_v, m0_out.at[pl.ds(base, BPW)])
    pltpu.sync_copy(m1_v, m1_out.at[pl.ds(base, BPW)])


def _sc_call(body, n_out, n_idx_scratch, extra_scratch):
    mesh = plsc.VectorSubcoreMesh(core_axis_name="c", subcore_axis_name="s")
    return pl.kernel(
        body,
        out_type=tuple(
            jax.ShapeDtypeStruct((B, D), jnp.float32) for _ in range(n_out)),
        mesh=mesh,
        scratch_types=(
            [pltpu.VMEM((NIDX, 128), jnp.int32) for _ in range(n_idx_scratch)]
            + extra_scratch),
        compiler_params=pltpu.CompilerParams(use_tc_tiling_on_sc=False),
    )


@jax.jit
def _sc_gather(idx, wg, wm):
    g, = _sc_call(_sc_gmf_body, 1, 2, [
        pltpu.VMEM((BPW, D), jnp.float32),
        pltpu.VMEM((BPW, D), jnp.float32),
        pltpu.SemaphoreType.DMA,
    ])(idx, wg)
    m0, m1 = _sc_call(_sc_mlp_body, 2, 2, [
        pltpu.VMEM((BPW, D), jnp.float32),
        pltpu.VMEM((BPW, D), jnp.float32),
        pltpu.SemaphoreType.DMA,
    ])(idx, wm)
    return g, m0, m1


BLK = 2048


def _tc_body(g_ref, m0_ref, m1_ref, w0a, w0b, b0, w1t, b1, w2t, b2,
             wcg, wch, bc, out_ref):
    f32 = jnp.float32
    h = jnp.dot(m0_ref[...], w0a[...], preferred_element_type=f32)
    h = h + jnp.dot(m1_ref[...], w0b[...], preferred_element_type=f32)
    h = jnp.maximum(h + b0[...], 0.0)
    h = jnp.maximum(jnp.dot(h, w1t[...], preferred_element_type=f32) + b1[...], 0.0)
    h = jnp.maximum(jnp.dot(h, w2t[...], preferred_element_type=f32) + b2[...], 0.0)
    logit = (jnp.dot(h, wch[...], preferred_element_type=f32)
             + jnp.dot(g_ref[...], wcg[...], preferred_element_type=f32)
             + bc[...])
    out_ref[...] = logit


@jax.jit
def _tc_mlp(g, m0, m1, w0a, w0b, b0, w1t, b1, w2t, b2, wcg, wch, bc):
    full = lambda shape: pl.BlockSpec(shape, lambda i: (0, 0))
    return pl.pallas_call(
        _tc_body,
        grid=(B // BLK,),
        in_specs=[
            pl.BlockSpec((BLK, D), lambda i: (i, 0)),
            pl.BlockSpec((BLK, D), lambda i: (i, 0)),
            pl.BlockSpec((BLK, D), lambda i: (i, 0)),
            full(w0a.shape), full(w0b.shape), full(b0.shape),
            full(w1t.shape), full(b1.shape),
            full(w2t.shape), full(b2.shape),
            full(wcg.shape), full(wch.shape), full(bc.shape),
        ],
        out_specs=pl.BlockSpec((BLK, 1), lambda i: (i, 0)),
        out_shape=jax.ShapeDtypeStruct((B, 1), jnp.float32),
    )(g, m0, m1, w0a, w0b, b0, w1t, b1, w2t, b2, wcg, wch, bc)


@jax.jit
def _pipeline(sparse_features, W_gmf, W_mlp, Wl0, bl0, Wl1, bl1, Wl2, bl2, Wc, bc):
    idx = sparse_features.astype(jnp.int32).T.reshape(2, NW, NIDX, 128)
    g, m0, m1 = _sc_gather(idx, W_gmf, W_mlp)
    return _tc_mlp(
        g, m0, m1,
        Wl0[:, :D].T, Wl0[:, D:].T, bl0.reshape(1, -1),
        Wl1.T, bl1.reshape(1, -1),
        Wl2.T, bl2.reshape(1, -1),
        Wc[:, :D].T, Wc[:, D:].T, bc.reshape(1, 1),
    )


def kernel(sparse_features, W_gmf, W_mlp, Wl0, bl0, Wl1, bl1, Wl2, bl2, Wc, bc):
    return _pipeline(sparse_features, W_gmf, W_mlp, Wl0, bl0, Wl1, bl1, Wl2,
                     bl2, Wc, bc)
